# baseline (speedup 1.0000x reference)
import jax
import jax.numpy as jnp
from jax import lax
from jax.experimental import pallas as pl
from jax.experimental.pallas import tpu as pltpu
from jax.experimental.pallas import tpu_sc as plsc

_N_EDGES = 320000
_D = 128
_L = 16
_NUM_REL = 500

_info = plsc.get_sparse_core_info()
_NC = _info.num_cores
_NS = _info.num_subcores
_NW = _NC * _NS
_EPW = _N_EDGES // _NW
_C = 80
_NCHUNK = _EPW // _C
_NCHUNK_P = 128

_GATHER_DNUMS = lax.GatherDimensionNumbers(
    offset_dims=(), collapsed_slice_dims=(0,), start_index_map=(0,))


def _rot(x, idx):
    return lax.gather(x, idx[:, None], _GATHER_DNUMS, slice_sizes=(1,),
                      mode=lax.GatherScatterMode.PROMISE_IN_BOUNDS)


def _hsum_all_lanes(x):
    for k in (8, 4, 2, 1):
        idx = (lax.iota(jnp.int32, _L) + k) & (_L - 1)
        x = x + _rot(x, idx)
    return x


def _tec_body(z_hbm, idx_hbm, rel_hbm, out_hbm,
              ib0, ib1, ib2, ib3, sr0, dr0, sr1, dr1, ob0, ob1, rel_v,
              si0, si1, si2, si3, semg0, semg1, semo0, semo1):
    wid = lax.axis_index("s") * _NC + lax.axis_index("c")
    base = wid * _EPW
    ibs = ((ib0, si0), (ib1, si1), (ib2, si2), (ib3, si3))
    bufs = ((sr0, dr0, ob0, semg0, semo0), (sr1, dr1, ob1, semg1, semo1))

    pltpu.sync_copy(rel_hbm, rel_v)

    def fire_idx(c, islot):
        ib, sem = islot

        @pl.when(c < _NCHUNK_P)
        def _():
            goff = (wid * _NCHUNK_P + c) * 3 * _C
            pltpu.async_copy(idx_hbm.at[pl.ds(goff, 3 * _C)], ib, sem)

    def fire_gather(c, buf, islot):
        ib, isem = islot
        sr, dr = buf[0], buf[1]

        @pl.when(c < _NCHUNK_P)
        def _():
            goff = (wid * _NCHUNK_P + c) * 3 * _C
            pltpu.make_async_copy(
                idx_hbm.at[pl.ds(goff, 3 * _C)], ib, isem).wait()
            pltpu.async_copy(z_hbm.at[ib.at[pl.ds(0, _C)]], sr, buf[3])
            pltpu.async_copy(z_hbm.at[ib.at[pl.ds(_C, _C)]], dr, buf[3])

    def compute(c, buf, islot):
        ib = islot[0]
        sr, dr, ob = buf[0], buf[1], buf[2]
        off = base + jnp.minimum(c, _NCHUNK - 1) * _C
        pltpu.make_async_copy(z_hbm.at[ib.at[pl.ds(0, _C)]], sr, buf[3]).wait()
        pltpu.make_async_copy(z_hbm.at[ib.at[pl.ds(_C, _C)]], dr, buf[3]).wait()

        for g in range(_C // _L):
            vec = jnp.zeros((_L,), jnp.float32)
            tvec = ib[pl.ds(2 * _C + g * _L, _L)]
            for l in range(_L):
                e = g * _L + l
                t = tvec[l]
                acc = jnp.zeros((_L,), jnp.float32)
                for j in range(_D // (2 * _L)):
                    sl = pl.ds(j * _L, _L)
                    sw, rw, dw = sr[e, sl], rel_v[t, sl], dr[e, sl]
                    s_hi = lax.bitcast_convert_type(sw, jnp.float32)
                    r_hi = lax.bitcast_convert_type(rw, jnp.float32)
                    d_hi = lax.bitcast_convert_type(dw, jnp.float32)
                    s_lo = lax.bitcast_convert_type(sw << 16, jnp.float32)
                    r_lo = lax.bitcast_convert_type(rw << 16, jnp.float32)
                    d_lo = lax.bitcast_convert_type(dw << 16, jnp.float32)
                    acc = acc + jnp.abs(s_lo + r_lo - d_lo)
                    acc = acc + jnp.abs(s_hi + r_hi - d_hi)
                lane = lax.iota(jnp.int32, _L) == l
                vec = jnp.where(lane, _hsum_all_lanes(acc), vec)
            ob[pl.ds(g * _L, _L)] = -vec

        pltpu.async_copy(ob, out_hbm.at[pl.ds(off, _C)], buf[4])

    def drain_out(c, buf):
        off = base + jnp.minimum(c, _NCHUNK - 1) * _C
        pltpu.make_async_copy(buf[2], out_hbm.at[pl.ds(off, _C)], buf[4]).wait()

    fire_idx(0, ibs[0])
    fire_idx(1, ibs[1])
    fire_idx(2, ibs[2])
    fire_gather(0, bufs[0], ibs[0])

    def pair_body(c0, ring):
        fire_gather(c0 + 1, bufs[1], ring[1])
        fire_idx(c0 + 3, ring[3])
        compute(c0, bufs[0], ring[0])
        fire_gather(c0 + 2, bufs[0], ring[2])
        fire_idx(c0 + 4, ring[0])
        compute(c0 + 1, bufs[1], ring[1])
        drain_out(c0, bufs[0])
        drain_out(c0 + 1, bufs[1])

    def quad(k, carry):
        c0 = k * 4
        pair_body(c0, (ibs[0], ibs[1], ibs[2], ibs[3]))
        pair_body(c0 + 2, (ibs[2], ibs[3], ibs[0], ibs[1]))
        return carry

    lax.fori_loop(0, _NCHUNK_P // 4, quad, 0)


_sc_call = pl.kernel(
    _tec_body,
    out_type=jax.ShapeDtypeStruct((_N_EDGES,), jnp.float32),
    mesh=plsc.VectorSubcoreMesh(core_axis_name="c", subcore_axis_name="s"),
    scratch_types=[
        pltpu.VMEM((3 * _C,), jnp.int32),
        pltpu.VMEM((3 * _C,), jnp.int32),
        pltpu.VMEM((3 * _C,), jnp.int32),
        pltpu.VMEM((3 * _C,), jnp.int32),
        pltpu.VMEM((_C, _D), jnp.int32),
        pltpu.VMEM((_C, _D), jnp.int32),
        pltpu.VMEM((_C, _D), jnp.int32),
        pltpu.VMEM((_C, _D), jnp.int32),
        pltpu.VMEM((_C,), jnp.float32),
        pltpu.VMEM((_C,), jnp.float32),
        pltpu.VMEM((_NUM_REL, _D // 2), jnp.int32),
        pltpu.SemaphoreType.DMA,
        pltpu.SemaphoreType.DMA,
        pltpu.SemaphoreType.DMA,
        pltpu.SemaphoreType.DMA,
        pltpu.SemaphoreType.DMA,
        pltpu.SemaphoreType.DMA,
        pltpu.SemaphoreType.DMA,
        pltpu.SemaphoreType.DMA,
    ],
)


@jax.jit
def kernel(z, edge_index, edge_type, rel_emb):
    idx_all = jnp.concatenate(
        [edge_index.astype(jnp.int32), edge_type.astype(jnp.int32)[None]], axis=0)
    idx_w = jnp.transpose(
        idx_all.reshape(3, _NW, _NCHUNK, _C), (1, 2, 0, 3))
    pad = jnp.repeat(idx_w[:, -1:], _NCHUNK_P - _NCHUNK, axis=1)
    idx_flat = jnp.concatenate([idx_w, pad], axis=1).reshape(-1)
    zi = lax.bitcast_convert_type(
        z.astype(jnp.bfloat16).reshape(-1, _D // 2, 2), jnp.int32)
    zi = jnp.concatenate([zi, zi], axis=1)
    ri = lax.bitcast_convert_type(
        rel_emb.astype(jnp.bfloat16).reshape(-1, _D // 2, 2), jnp.int32)
    return _sc_call(zi, idx_flat, ri)


# speedup vs baseline: 1.7076x; 1.7076x over previous
import jax
import jax.numpy as jnp
from jax import lax
from jax.experimental import pallas as pl
from jax.experimental.pallas import tpu as pltpu
from jax.experimental.pallas import tpu_sc as plsc

_N_EDGES = 320000
_D = 128
_L = 16
_NUM_REL = 500

_info = plsc.get_sparse_core_info()
_NC = _info.num_cores
_NS = _info.num_subcores
_NW = _NC * _NS
_EPW = _N_EDGES // _NW
_C = 80
_NCHUNK = _EPW // _C
_NCHUNK_P = 128

_GATHER_DNUMS = lax.GatherDimensionNumbers(
    offset_dims=(), collapsed_slice_dims=(0,), start_index_map=(0,))


def _rot(x, idx):
    return lax.gather(x, idx[:, None], _GATHER_DNUMS, slice_sizes=(1,),
                      mode=lax.GatherScatterMode.PROMISE_IN_BOUNDS)


def _hsum_all_lanes(x):
    for k in (8, 4, 2, 1):
        idx = (lax.iota(jnp.int32, _L) + k) & (_L - 1)
        x = x + _rot(x, idx)
    return x


def _tec_body(z_hbm, idx_hbm, rel_hbm, out_hbm,
              ib0, ib1, ib2, ib3, sr0, dr0, sr1, dr1, ob0, ob1, rel_v,
              si0, si1, si2, si3, semg0, semg1, semo0, semo1):
    wid = lax.axis_index("s") * _NC + lax.axis_index("c")
    base = wid * _EPW
    ibs = ((ib0, si0), (ib1, si1), (ib2, si2), (ib3, si3))
    bufs = ((sr0, dr0, ob0, semg0, semo0), (sr1, dr1, ob1, semg1, semo1))

    pltpu.sync_copy(rel_hbm, rel_v)

    def fire_idx(c, islot):
        ib, sem = islot

        @pl.when(c < _NCHUNK_P)
        def _():
            goff = (wid * _NCHUNK_P + c) * 3 * _C
            pltpu.async_copy(idx_hbm.at[pl.ds(goff, 3 * _C)], ib, sem)

    def fire_gather(c, buf, islot):
        ib, isem = islot
        sr, dr = buf[0], buf[1]

        @pl.when(c < _NCHUNK_P)
        def _():
            goff = (wid * _NCHUNK_P + c) * 3 * _C
            pltpu.make_async_copy(
                idx_hbm.at[pl.ds(goff, 3 * _C)], ib, isem).wait()
            pltpu.async_copy(z_hbm.at[ib.at[pl.ds(0, _C)]], sr, buf[3])
            pltpu.async_copy(z_hbm.at[ib.at[pl.ds(_C, _C)]], dr, buf[3])

    def compute(c, buf, islot):
        ib = islot[0]
        sr, dr, ob = buf[0], buf[1], buf[2]
        off = base + jnp.minimum(c, _NCHUNK - 1) * _C
        pltpu.make_async_copy(z_hbm.at[ib.at[pl.ds(0, _C)]], sr, buf[3]).wait()
        pltpu.make_async_copy(z_hbm.at[ib.at[pl.ds(_C, _C)]], dr, buf[3]).wait()

        def group(g, carry2):
            vec = jnp.zeros((_L,), jnp.float32)
            tvec = ib[pl.ds(2 * _C + g * _L, _L)]
            for l in range(_L):
                e = g * _L + l
                t = tvec[l]
                acc = jnp.zeros((_L,), jnp.float32)
                for j in range(_D // (2 * _L)):
                    sl = pl.ds(j * _L, _L)
                    sw, rw, dw = sr[e, sl], rel_v[t, sl], dr[e, sl]
                    s_hi = lax.bitcast_convert_type(sw, jnp.float32)
                    r_hi = lax.bitcast_convert_type(rw, jnp.float32)
                    d_hi = lax.bitcast_convert_type(dw, jnp.float32)
                    s_lo = lax.bitcast_convert_type(sw << 16, jnp.float32)
                    r_lo = lax.bitcast_convert_type(rw << 16, jnp.float32)
                    d_lo = lax.bitcast_convert_type(dw << 16, jnp.float32)
                    acc = acc + jnp.abs(s_lo + r_lo - d_lo)
                    acc = acc + jnp.abs(s_hi + r_hi - d_hi)
                lane = lax.iota(jnp.int32, _L) == l
                vec = jnp.where(lane, _hsum_all_lanes(acc), vec)
            ob[pl.ds(g * _L, _L)] = -vec
            return carry2

        lax.fori_loop(0, _C // _L, group, 0)
        pltpu.async_copy(ob, out_hbm.at[pl.ds(off, _C)], buf[4])

    def drain_out(c, buf):
        off = base + jnp.minimum(c, _NCHUNK - 1) * _C
        pltpu.make_async_copy(buf[2], out_hbm.at[pl.ds(off, _C)], buf[4]).wait()

    fire_idx(0, ibs[0])
    fire_idx(1, ibs[1])
    fire_idx(2, ibs[2])
    fire_gather(0, bufs[0], ibs[0])

    def pair_body(c0, ring):
        fire_gather(c0 + 1, bufs[1], ring[1])
        fire_idx(c0 + 3, ring[3])
        compute(c0, bufs[0], ring[0])
        fire_gather(c0 + 2, bufs[0], ring[2])
        fire_idx(c0 + 4, ring[0])
        compute(c0 + 1, bufs[1], ring[1])
        drain_out(c0, bufs[0])
        drain_out(c0 + 1, bufs[1])

    def quad(k, carry):
        c0 = k * 4
        pair_body(c0, (ibs[0], ibs[1], ibs[2], ibs[3]))
        pair_body(c0 + 2, (ibs[2], ibs[3], ibs[0], ibs[1]))
        return carry

    lax.fori_loop(0, _NCHUNK_P // 4, quad, 0)


_sc_call = pl.kernel(
    _tec_body,
    out_type=jax.ShapeDtypeStruct((_N_EDGES,), jnp.float32),
    mesh=plsc.VectorSubcoreMesh(core_axis_name="c", subcore_axis_name="s"),
    scratch_types=[
        pltpu.VMEM((3 * _C,), jnp.int32),
        pltpu.VMEM((3 * _C,), jnp.int32),
        pltpu.VMEM((3 * _C,), jnp.int32),
        pltpu.VMEM((3 * _C,), jnp.int32),
        pltpu.VMEM((_C, _D), jnp.int32),
        pltpu.VMEM((_C, _D), jnp.int32),
        pltpu.VMEM((_C, _D), jnp.int32),
        pltpu.VMEM((_C, _D), jnp.int32),
        pltpu.VMEM((_C,), jnp.float32),
        pltpu.VMEM((_C,), jnp.float32),
        pltpu.VMEM((_NUM_REL, _D // 2), jnp.int32),
        pltpu.SemaphoreType.DMA,
        pltpu.SemaphoreType.DMA,
        pltpu.SemaphoreType.DMA,
        pltpu.SemaphoreType.DMA,
        pltpu.SemaphoreType.DMA,
        pltpu.SemaphoreType.DMA,
        pltpu.SemaphoreType.DMA,
        pltpu.SemaphoreType.DMA,
    ],
)


@jax.jit
def kernel(z, edge_index, edge_type, rel_emb):
    idx_all = jnp.concatenate(
        [edge_index.astype(jnp.int32), edge_type.astype(jnp.int32)[None]], axis=0)
    idx_w = jnp.transpose(
        idx_all.reshape(3, _NW, _NCHUNK, _C), (1, 2, 0, 3))
    pad = jnp.repeat(idx_w[:, -1:], _NCHUNK_P - _NCHUNK, axis=1)
    idx_flat = jnp.concatenate([idx_w, pad], axis=1).reshape(-1)
    zi = lax.bitcast_convert_type(
        z.astype(jnp.bfloat16).reshape(-1, _D // 2, 2), jnp.int32)
    zi = jnp.concatenate([zi, zi], axis=1)
    ri = lax.bitcast_convert_type(
        rel_emb.astype(jnp.bfloat16).reshape(-1, _D // 2, 2), jnp.int32)
    return _sc_call(zi, idx_flat, ri)
